# baseline (speedup 1.0000x reference)
import functools

import jax
import jax.numpy as jnp
from jax import lax
from jax.experimental import pallas as pl
from jax.experimental.pallas import tpu as pltpu
from jax.experimental.pallas import tpu_sc as plsc

B = 4096
L = 200
D = 64

NC = 2
NS = 16
NW = NC * NS

ROWS = B * L
RPW = ROWS // NW
CHUNK = 1024
NCHUNK = RPW // CHUNK
G = CHUNK // 128


def _make_kernel():
    mesh = plsc.VectorSubcoreMesh(core_axis_name="c", subcore_axis_name="s")

    @functools.partial(
        pl.kernel,
        mesh=mesh,
        compiler_params=pltpu.CompilerParams(use_tc_tiling_on_sc=False),
        out_type=jax.ShapeDtypeStruct((ROWS, D), jnp.float32),
        scratch_types=[
            pltpu.VMEM((G, 128), jnp.int32),
            pltpu.VMEM((CHUNK, D), jnp.float32),
            pltpu.VMEM((L, D), jnp.float32),
            pltpu.SemaphoreType.DMA,
        ],
    )
    def emb_kernel(idx_hbm, tok_hbm, pos_hbm, out_hbm, idx_v, rows_v, pos_v, sem):
        wid = lax.axis_index("s") * NC + lax.axis_index("c")
        base = wid * RPW

        pltpu.sync_copy(pos_hbm.at[pl.ds(0, L)], pos_v)

        def chunk_body(c, carry):
            cbase = base + c * CHUNK
            goff = pl.multiple_of(cbase // 128, 8)
            pltpu.sync_copy(idx_hbm.at[pl.ds(goff, G)], idx_v)
            descs = []
            for j in range(G):
                descs.append(
                    pltpu.async_copy(
                        tok_hbm.at[idx_v.at[j]],
                        rows_v.at[pl.ds(j * 128, 128)],
                        sem,
                    )
                )
            for d in descs:
                d.wait()

            def row_body(r, rcarry):
                pr = lax.rem(cbase + r, L)
                for k in range(D // 16):
                    sl = pl.ds(k * 16, 16)
                    rows_v[r, sl] = rows_v[r, sl] + pos_v[pr, sl]
                return rcarry

            lax.fori_loop(0, CHUNK, row_body, 0)

            pltpu.sync_copy(rows_v, out_hbm.at[pl.ds(cbase, CHUNK)])
            return carry

        lax.fori_loop(0, NCHUNK, chunk_body, 0)

    return emb_kernel


_emb = _make_kernel()


@jax.jit
def kernel(x, token_table, pos_table):
    idx = x.reshape(ROWS // 128, 128)
    out = _emb(idx, token_table, pos_table)
    return out.reshape(B, L, D)


# speedup vs baseline: 1.0665x; 1.0665x over previous
import functools

import jax
import jax.numpy as jnp
from jax import lax
from jax.experimental import pallas as pl
from jax.experimental.pallas import tpu as pltpu
from jax.experimental.pallas import tpu_sc as plsc

B = 4096
L = 200
D = 64

NC = 2
NS = 16
NW = NC * NS

ROWS = B * L
RPW = ROWS // NW
CHUNK = 512
NCHUNK = RPW // CHUNK
G = CHUNK // 128
IDXROWS = RPW // 128


def _make_kernel():
    mesh = plsc.VectorSubcoreMesh(core_axis_name="c", subcore_axis_name="s")

    @functools.partial(
        pl.kernel,
        mesh=mesh,
        compiler_params=pltpu.CompilerParams(use_tc_tiling_on_sc=False),
        out_type=jax.ShapeDtypeStruct((ROWS, D), jnp.float32),
        scratch_types=[
            pltpu.VMEM((IDXROWS, 128), jnp.int32),
            pltpu.VMEM((CHUNK, D), jnp.float32),
            pltpu.VMEM((CHUNK, D), jnp.float32),
            pltpu.VMEM((L, D), jnp.float32),
            pltpu.SemaphoreType.DMA,
            pltpu.SemaphoreType.DMA,
            pltpu.SemaphoreType.DMA,
            pltpu.SemaphoreType.DMA,
        ],
    )
    def emb_kernel(idx_hbm, tok_hbm, pos_hbm, out_hbm,
                   idx_v, rows0, rows1, pos_v, gsem0, gsem1, osem0, osem1):
        wid = lax.axis_index("s") * NC + lax.axis_index("c")
        base = wid * RPW
        rows = (rows0, rows1)
        gsem = (gsem0, gsem1)
        osem = (osem0, osem1)

        pltpu.sync_copy(pos_hbm.at[pl.ds(0, L)], pos_v)
        ioff = pl.multiple_of(wid * IDXROWS, 8)
        pltpu.sync_copy(idx_hbm.at[pl.ds(ioff, IDXROWS)], idx_v)

        def start_gathers(i, b):
            for k in range(G):
                pltpu.async_copy(
                    tok_hbm.at[idx_v.at[i * G + k]],
                    rows[b].at[pl.ds(k * 128, 128)],
                    gsem[b],
                )

        def wait_gathers(b):
            pltpu.make_async_copy(
                out_hbm.at[pl.ds(0, CHUNK)], rows[b], gsem[b]
            ).wait()

        def wait_store(b):
            pltpu.make_async_copy(
                rows[b], out_hbm.at[pl.ds(0, CHUNK)], osem[b]
            ).wait()

        def add_pos(i, b):
            p0 = lax.rem(i * CHUNK, L)

            def row_body(r, p):
                for k in range(D // 16):
                    sl = pl.ds(k * 16, 16)
                    rows[b][r, sl] = rows[b][r, sl] + pos_v[p, sl]
                p = p + 1
                return lax.select(p == L, 0, p)

            lax.fori_loop(0, CHUNK, row_body, p0, unroll=4)

        start_gathers(0, 0)

        def pair_body(g, carry):
            for b in range(2):
                i = 2 * g + b
                not_first = i >= 1
                pl.when(not_first)(lambda: wait_store(1 - b))
                pl.when(i + 1 < NCHUNK)(lambda: start_gathers(i + 1, 1 - b))
                wait_gathers(b)
                add_pos(i, b)
                pltpu.async_copy(
                    rows[b],
                    out_hbm.at[pl.ds(base + i * CHUNK, CHUNK)],
                    osem[b],
                )
            return carry

        lax.fori_loop(0, NCHUNK // 2, pair_body, 0)
        wait_store(1)

    return emb_kernel


_emb = _make_kernel()


@jax.jit
def kernel(x, token_table, pos_table):
    idx = x.reshape(ROWS // 128, 128)
    out = _emb(idx, token_table, pos_table)
    return out.reshape(B, L, D)


# speedup vs baseline: 1.1827x; 1.1090x over previous
import functools

import jax
import jax.numpy as jnp
from jax import lax
from jax.experimental import pallas as pl
from jax.experimental.pallas import tpu as pltpu
from jax.experimental.pallas import tpu_sc as plsc

B = 4096
L = 200
D = 64
VOCAB = 1000000

NC = 2
NS = 16
NW = NC * NS

ROWS = B * L
RPW = ROWS // NW
C = 128
NCH = RPW // C


def _make_kernel():
    mesh = plsc.VectorSubcoreMesh(core_axis_name="c", subcore_axis_name="s")

    @functools.partial(
        pl.kernel,
        mesh=mesh,
        out_type=jax.ShapeDtypeStruct((ROWS, D), jnp.float32),
        scratch_types=[
            pltpu.VMEM((RPW,), jnp.int32),
            pltpu.VMEM((RPW,), jnp.int32),
            pltpu.VMEM((L * D,), jnp.float32),
            pltpu.VMEM((C, 128), jnp.float32),
            pltpu.VMEM((C, 128), jnp.float32),
            pltpu.VMEM((C, D), jnp.float32),
            pltpu.VMEM((C, D), jnp.float32),
            pltpu.SemaphoreType.DMA,
            pltpu.SemaphoreType.DMA,
            pltpu.SemaphoreType.DMA,
            pltpu.SemaphoreType.DMA,
        ],
    )
    def emb_kernel(hx_hbm, par_hbm, tok2_hbm, posf_hbm, out_hbm,
                   idx_v, par_v, pos_v, g0, g1, o0, o1,
                   gsem0, gsem1, osem0, osem1):
        wid = lax.axis_index("s") * NC + lax.axis_index("c")
        base = wid * RPW
        g = (g0, g1)
        o = (o0, o1)
        gsem = (gsem0, gsem1)
        osem = (osem0, osem1)

        pltpu.sync_copy(hx_hbm.at[pl.ds(base, RPW)], idx_v)
        pltpu.sync_copy(par_hbm.at[pl.ds(base, RPW)], par_v)
        pltpu.sync_copy(posf_hbm, pos_v)

        def start_gather(i, b):
            ioff = pl.multiple_of(i * C, 8)
            pltpu.async_copy(
                tok2_hbm.at[idx_v.at[pl.ds(ioff, C)]], g[b], gsem[b]
            )

        def wait_gather(b):
            pltpu.make_async_copy(tok2_hbm.at[pl.ds(0, C)], g[b], gsem[b]).wait()

        def wait_store(b):
            pltpu.make_async_copy(o[b], out_hbm.at[pl.ds(0, C)], osem[b]).wait()

        def select_add(i, b):
            p0 = lax.rem(i * C, L)

            def grp_body(q, carry):
                rbase = q * 16
                pv = par_v[pl.ds(pl.multiple_of(i * C + rbase, 8), 16)]
                for j in range(16):
                    r = rbase + j
                    off = pl.multiple_of(pv[j], 8)
                    pr = p0 + r
                    pr = lax.select(pr >= L, pr - L, pr)
                    pb = pl.multiple_of(pr * D, 8)
                    for k in range(D // 16):
                        o[b][r, pl.ds(k * 16, 16)] = (
                            g[b][r, pl.ds(off + k * 16, 16)]
                            + pos_v[pl.ds(pb + k * 16, 16)]
                        )
                return carry

            lax.fori_loop(0, C // 16, grp_body, 0)

        start_gather(0, 0)

        def pair_body(h, carry):
            for b in range(2):
                i = 2 * h + b
                pl.when(i >= 1)(lambda: wait_store(1 - b))
                pl.when(i + 1 < NCH)(lambda: start_gather(i + 1, 1 - b))
                wait_gather(b)
                select_add(i, b)
                pltpu.async_copy(
                    o[b], out_hbm.at[pl.ds(base + i * C, C)], osem[b]
                )
            return carry

        lax.fori_loop(0, NCH // 2, pair_body, 0)
        wait_store(1)

    return emb_kernel


_emb = _make_kernel()


@jax.jit
def kernel(x, token_table, pos_table):
    xf = x.reshape(-1)
    hx = xf >> 1
    par = (xf & 1) << 6
    tok2 = token_table.reshape(VOCAB // 2, 128)
    posf = pos_table[:L].reshape(-1)
    out = _emb(hx, par, tok2, posf)
    return out.reshape(B, L, D)


# speedup vs baseline: 1.2593x; 1.0648x over previous
import functools

import jax
import jax.numpy as jnp
from jax import lax
from jax.experimental import pallas as pl
from jax.experimental.pallas import tpu as pltpu
from jax.experimental.pallas import tpu_sc as plsc

B = 4096
L = 200
D = 64

NC = 2
NS = 16
NW = NC * NS

ROWS = B * L
RPW = ROWS // NW
C = 256
NCH = RPW // C
G = C // 128


def _make_kernel():
    mesh = plsc.VectorSubcoreMesh(core_axis_name="c", subcore_axis_name="s")

    @functools.partial(
        pl.kernel,
        mesh=mesh,
        compiler_params=pltpu.CompilerParams(use_tc_tiling_on_sc=False),
        out_type=jax.ShapeDtypeStruct((ROWS, 2 * D), jnp.float32),
        scratch_types=[
            pltpu.VMEM((RPW,), jnp.int32),
            pltpu.VMEM((L * D,), jnp.float32),
            pltpu.VMEM((C, D), jnp.float32),
            pltpu.VMEM((C, D), jnp.float32),
            pltpu.VMEM((C, D), jnp.float32),
            pltpu.VMEM((C, D), jnp.float32),
            pltpu.SemaphoreType.DMA,
            pltpu.SemaphoreType.DMA,
            pltpu.SemaphoreType.DMA,
            pltpu.SemaphoreType.DMA,
        ],
    )
    def emb_kernel(xf_hbm, tok_hbm, posf_hbm, out_hbm,
                   idx_v, pos_v, g0, g1, o0, o1,
                   gsem0, gsem1, osem0, osem1):
        wid = lax.axis_index("s") * NC + lax.axis_index("c")
        base = wid * RPW
        g = (g0, g1)
        o = (o0, o1)
        gsem = (gsem0, gsem1)
        osem = (osem0, osem1)

        pltpu.sync_copy(xf_hbm.at[pl.ds(base, RPW)], idx_v)
        pltpu.sync_copy(posf_hbm, pos_v)

        def start_gathers(i, b):
            for k in range(G):
                ioff = pl.multiple_of(i * C + k * 128, 8)
                pltpu.async_copy(
                    tok_hbm.at[idx_v.at[pl.ds(ioff, 128)]],
                    g[b].at[pl.ds(k * 128, 128)],
                    gsem[b],
                )

        def wait_gathers(b):
            pltpu.make_async_copy(tok_hbm.at[pl.ds(0, C)], g[b], gsem[b]).wait()

        def wait_store(b):
            pltpu.make_async_copy(
                o[b], out_hbm.at[pl.ds(0, C), pl.ds(0, D)], osem[b]
            ).wait()

        def add_pos(i, b):
            p0 = lax.rem(i * C, L)

            def row_body(r, carry):
                p = p0 + r
                p = lax.select(p >= L, p - L, p)
                p = lax.select(p >= L, p - L, p)
                pb = pl.multiple_of(p * D, 8)
                for k in range(D // 16):
                    o[b][r, pl.ds(k * 16, 16)] = (
                        g[b][r, pl.ds(k * 16, 16)]
                        + pos_v[pl.ds(pb + k * 16, 16)]
                    )
                return carry

            lax.fori_loop(0, C, row_body, 0, unroll=8)

        start_gathers(0, 0)

        def pair_body(h, carry):
            for b in range(2):
                i = 2 * h + b
                pl.when(i >= 1)(lambda: wait_store(1 - b))
                pl.when(i + 1 < NCH)(lambda: start_gathers(i + 1, 1 - b))
                wait_gathers(b)
                add_pos(i, b)
                pltpu.async_copy(
                    o[b],
                    out_hbm.at[pl.ds(base + i * C, C), pl.ds(0, D)],
                    osem[b],
                )
            return carry

        lax.fori_loop(0, NCH // 2, pair_body, 0)
        wait_store(1)

    return emb_kernel


_emb = _make_kernel()


@jax.jit
def kernel(x, token_table, pos_table):
    xf = x.reshape(-1)
    posf = pos_table[:L].reshape(-1)
    out2 = _emb(xf, token_table, posf)
    return out2[:, :D].reshape(B, L, D)


# speedup vs baseline: 2.1784x; 1.7298x over previous
import functools

import jax
import jax.numpy as jnp
from jax import lax
from jax.experimental import pallas as pl
from jax.experimental.pallas import tpu as pltpu
from jax.experimental.pallas import tpu_sc as plsc

B = 4096
L = 200
D = 64

NC = 2
NS = 16
NW = NC * NS

ROWS = B * L
RPW = ROWS // NW
C = 256
NCH = RPW // C
G = C // 128


def _make_kernel():
    mesh = plsc.VectorSubcoreMesh(core_axis_name="c", subcore_axis_name="s")

    @functools.partial(
        pl.kernel,
        mesh=mesh,
        compiler_params=pltpu.CompilerParams(use_tc_tiling_on_sc=False),
        out_type=jax.ShapeDtypeStruct((ROWS, 2 * D), jnp.float32),
        scratch_types=[
            pltpu.VMEM((RPW,), jnp.int32),
            pltpu.VMEM((L * D,), jnp.float32),
            pltpu.VMEM((C, D), jnp.float32),
            pltpu.VMEM((C, D), jnp.float32),
            pltpu.VMEM((C, D), jnp.float32),
            pltpu.VMEM((C, D), jnp.float32),
            pltpu.SemaphoreType.DMA,
            pltpu.SemaphoreType.DMA,
            pltpu.SemaphoreType.DMA,
            pltpu.SemaphoreType.DMA,
        ],
    )
    def emb_kernel(xf_hbm, tok_hbm, posf_hbm, out_hbm,
                   idx_v, pos_v, g0, g1, o0, o1,
                   gsem0, gsem1, osem0, osem1):
        wid = lax.axis_index("s") * NC + lax.axis_index("c")
        base = wid * RPW
        g = (g0, g1)
        o = (o0, o1)
        gsem = (gsem0, gsem1)
        osem = (osem0, osem1)

        pltpu.sync_copy(xf_hbm.at[pl.ds(base, RPW)], idx_v)
        pltpu.sync_copy(posf_hbm, pos_v)

        def start_gathers(i, b):
            for k in range(G):
                ioff = pl.multiple_of(i * C + k * 128, 8)
                pltpu.async_copy(
                    tok_hbm.at[idx_v.at[pl.ds(ioff, 128)]],
                    g[b].at[pl.ds(k * 128, 128)],
                    gsem[b],
                )

        def wait_gathers(b):
            pltpu.make_async_copy(tok_hbm.at[pl.ds(0, C)], g[b], gsem[b]).wait()

        def wait_store(b):
            pltpu.make_async_copy(
                o[b], out_hbm.at[pl.ds(0, C), pl.ds(0, D)], osem[b]
            ).wait()

        def add_pos(i, b):
            p0 = lax.rem(i * C, L)

            @plsc.parallel_loop(0, C, 1, unroll=8)
            def _row_body(r):
                p = p0 + r
                p = lax.select(p >= L, p - L, p)
                p = lax.select(p >= L, p - L, p)
                pb = pl.multiple_of(p * D, 8)
                for k in range(D // 16):
                    o[b][r, pl.ds(k * 16, 16)] = (
                        g[b][r, pl.ds(k * 16, 16)]
                        + pos_v[pl.ds(pb + k * 16, 16)]
                    )

        start_gathers(0, 0)

        def pair_body(h, carry):
            for b in range(2):
                i = 2 * h + b
                pl.when(i >= 1)(lambda: wait_store(1 - b))
                pl.when(i + 1 < NCH)(lambda: start_gathers(i + 1, 1 - b))
                wait_gathers(b)
                add_pos(i, b)
                pltpu.async_copy(
                    o[b],
                    out_hbm.at[pl.ds(base + i * C, C), pl.ds(0, D)],
                    osem[b],
                )
            return carry

        lax.fori_loop(0, NCH // 2, pair_body, 0)
        wait_store(1)

    return emb_kernel


_emb = _make_kernel()


@jax.jit
def kernel(x, token_table, pos_table):
    xf = x.reshape(-1)
    posf = pos_table[:L].reshape(-1)
    out2 = _emb(xf, token_table, posf)
    return out2[:, :D].reshape(B, L, D)
